# Initial kernel scaffold; baseline (speedup 1.0000x reference)
import jax
import jax.numpy as jnp
from jax.experimental import pallas as pl


def kernel(support_set, query_set, support_labels, n_way, W, b):
    raise NotImplementedError("write your pallas kernel here")


# speedup vs baseline: 8.8356x; 8.8356x over previous
import functools

import jax
import jax.numpy as jnp
from jax import lax
from jax.experimental import pallas as pl
from jax.experimental.pallas import tpu as pltpu
from jax.experimental.pallas import tpu_sc as plsc

N_SUP = 65536
N_QRY = 8192
D = 128
C = 512
L = 16
CH = 128
QB = 1024

_HI = lax.Precision.HIGHEST


def _make_sc_segment_sum():
    mesh = plsc.VectorSubcoreMesh(core_axis_name="c", subcore_axis_name="s")
    nc, ns = mesh.num_cores, mesh.num_subcores
    nw = nc * ns
    rows_per_worker = N_SUP // nw
    nchunk = rows_per_worker // CH
    rows_per_tile = C // ns

    @functools.partial(
        pl.kernel,
        out_type=(
            jax.ShapeDtypeStruct((nc, C, D), jnp.float32),
            jax.ShapeDtypeStruct((nc, C, D), jnp.float32),
        ),
        mesh=mesh,
        scratch_types=[
            pltpu.VMEM((CH, D), jnp.float32),
            pltpu.VMEM((CH,), jnp.int32),
            pltpu.VMEM((CH, D), jnp.float32),
            pltpu.VMEM((rows_per_tile, D), jnp.float32),
            pltpu.VMEM_SHARED((C, D), jnp.float32),
            pltpu.VMEM_SHARED((C, D), jnp.float32),
        ],
    )
    def seg(support_hbm, labels_hbm, sums_out, counts_out,
            rows_v, labels_v, ones_v, stage_v, ssum, scnt):
        cid = lax.axis_index("c")
        sid = lax.axis_index("s")
        wid = sid * nc + cid

        zeros16 = jnp.zeros((L,), jnp.float32)
        ones16 = jnp.ones((L,), jnp.float32)

        def fill_ones(k, _):
            ones_v[k // (D // L), pl.ds((k % (D // L)) * L, L)] = ones16
            return 0
        lax.fori_loop(0, CH * (D // L), fill_ones, 0)

        def fill_zero(k, _):
            stage_v[k // (D // L), pl.ds((k % (D // L)) * L, L)] = zeros16
            return 0
        lax.fori_loop(0, rows_per_tile * (D // L), fill_zero, 0)

        rowbase = sid * rows_per_tile
        pltpu.sync_copy(stage_v, ssum.at[pl.ds(rowbase, rows_per_tile)])
        pltpu.sync_copy(stage_v, scnt.at[pl.ds(rowbase, rows_per_tile)])
        plsc.subcore_barrier()

        base = wid * rows_per_worker

        def chunk(t, _):
            off = base + t * CH
            pltpu.sync_copy(support_hbm.at[pl.ds(off, CH)], rows_v)
            pltpu.sync_copy(labels_hbm.at[pl.ds(off, CH)], labels_v)
            pltpu.sync_copy(rows_v, ssum.at[labels_v], add=True)
            pltpu.sync_copy(ones_v, scnt.at[labels_v], add=True)
            return 0
        lax.fori_loop(0, nchunk, chunk, 0)

        plsc.subcore_barrier()

        pltpu.sync_copy(ssum.at[pl.ds(rowbase, rows_per_tile)], stage_v)
        pltpu.sync_copy(stage_v, sums_out.at[cid, pl.ds(rowbase, rows_per_tile)])
        pltpu.sync_copy(scnt.at[pl.ds(rowbase, rows_per_tile)], stage_v)
        pltpu.sync_copy(stage_v, counts_out.at[cid, pl.ds(rowbase, rows_per_tile)])

    return seg


def _tc_body(ps_ref, pc_ref, q_ref, w_ref, b_ref, out_ref, sp_ref, row_ref):
    i = pl.program_id(0)

    @pl.when(i == 0)
    def _():
        s = ps_ref[0] + ps_ref[1]
        sp = jnp.dot(s, w_ref[:], preferred_element_type=jnp.float32,
                     precision=_HI)
        sp_ref[:] = sp
        csum = pc_ref[0] + pc_ref[1]
        ones_row = jnp.ones((1, D), jnp.float32)
        cnt = lax.dot_general(ones_row, csum, (((1,), (1,)), ((), ())),
                              preferred_element_type=jnp.float32,
                              precision=_HI) * (1.0 / D)
        nonempty = cnt > 0.5
        inv = jnp.where(nonempty, 1.0 / jnp.where(nonempty, cnt, 1.0), 0.0)
        msk = jnp.where(nonempty, 1.0, 0.0)
        b_row = b_ref[:]
        sn = lax.dot_general(ones_row, sp * sp, (((1,), (1,)), ((), ())),
                             preferred_element_type=jnp.float32,
                             precision=_HI)
        sb = lax.dot_general(b_row, sp, (((1,), (1,)), ((), ())),
                             preferred_element_type=jnp.float32,
                             precision=_HI)
        bb = jnp.sum(b_row * b_row)
        row_ref[0, :] = inv[0]
        row_ref[1, :] = msk[0]
        row_ref[2, :] = (sn * inv * inv + 2.0 * sb * inv + bb * msk)[0]

    q = q_ref[:]
    e = jnp.dot(q, w_ref[:], preferred_element_type=jnp.float32,
                precision=_HI) + b_ref[:]
    qn = jnp.sum(e * e, axis=1, keepdims=True)
    eb = jnp.sum(e * b_ref[:], axis=1, keepdims=True)
    a = lax.dot_general(e, sp_ref[:], (((1,), (1,)), ((), ())),
                        preferred_element_type=jnp.float32,
                        precision=_HI)
    inv = row_ref[0, :][None, :]
    msk = row_ref[1, :][None, :]
    pn = row_ref[2, :][None, :]
    d2 = qn + pn - 2.0 * (a * inv + eb * msk)
    out_ref[:] = -jnp.sqrt(jnp.maximum(d2, 0.0))


def _tc_distance(psums, pcounts, query, W, b2d):
    nc = psums.shape[0]
    grid = N_QRY // QB
    return pl.pallas_call(
        _tc_body,
        grid=(grid,),
        in_specs=[
            pl.BlockSpec((nc, C, D), lambda i: (0, 0, 0)),
            pl.BlockSpec((nc, C, D), lambda i: (0, 0, 0)),
            pl.BlockSpec((QB, D), lambda i: (i, 0)),
            pl.BlockSpec((D, D), lambda i: (0, 0)),
            pl.BlockSpec((1, D), lambda i: (0, 0)),
        ],
        out_specs=pl.BlockSpec((QB, C), lambda i: (i, 0)),
        out_shape=jax.ShapeDtypeStruct((N_QRY, C), jnp.float32),
        scratch_shapes=[
            pltpu.VMEM((C, D), jnp.float32),
            pltpu.VMEM((8, C), jnp.float32),
        ],
    )(psums, pcounts, query, W, b2d)


def kernel(support_set, query_set, support_labels, n_way, W, b):
    psums, pcounts = _make_sc_segment_sum()(support_set, support_labels)
    return _tc_distance(psums, pcounts, query_set, W, b.reshape(1, D))


# speedup vs baseline: 10.0962x; 1.1427x over previous
import functools

import jax
import jax.numpy as jnp
from jax import lax
from jax.experimental import pallas as pl
from jax.experimental.pallas import tpu as pltpu
from jax.experimental.pallas import tpu_sc as plsc

N_SUP = 65536
N_QRY = 8192
D = 128
C = 512
L = 16
CH = 128
QB = 1024

_HI = lax.Precision.HIGHEST


def _make_sc_segment_sum():
    mesh = plsc.VectorSubcoreMesh(core_axis_name="c", subcore_axis_name="s")
    nc, ns = mesh.num_cores, mesh.num_subcores
    nw = nc * ns
    rows_per_worker = N_SUP // nw
    nchunk = rows_per_worker // CH
    rows_per_tile = C // ns

    @functools.partial(
        pl.kernel,
        out_type=(
            jax.ShapeDtypeStruct((nc, C, D), jnp.float32),
            jax.ShapeDtypeStruct((nc, C), jnp.float32),
        ),
        mesh=mesh,
        scratch_types=[
            pltpu.VMEM((CH, D), jnp.float32),
            pltpu.VMEM((CH,), jnp.int32),
            pltpu.VMEM((CH,), jnp.float32),
            pltpu.VMEM((rows_per_tile, D), jnp.float32),
            pltpu.VMEM((C,), jnp.float32),
            pltpu.VMEM_SHARED((C, D), jnp.float32),
            pltpu.VMEM_SHARED((C,), jnp.float32),
        ],
    )
    def seg(support_hbm, labels_hbm, sums_out, counts_out,
            rows_v, labels_v, ones_v, stage_v, cstage_v, ssum, scnt):
        cid = lax.axis_index("c")
        sid = lax.axis_index("s")
        wid = sid * nc + cid

        zeros16 = jnp.zeros((L,), jnp.float32)
        ones16 = jnp.ones((L,), jnp.float32)

        def fill_ones(k, _):
            ones_v[pl.ds(k * L, L)] = ones16
            return 0
        lax.fori_loop(0, CH // L, fill_ones, 0)

        def fill_zero(k, _):
            stage_v[k // (D // L), pl.ds((k % (D // L)) * L, L)] = zeros16
            return 0
        lax.fori_loop(0, rows_per_tile * (D // L), fill_zero, 0)

        def fill_czero(k, _):
            cstage_v[pl.ds(k * L, L)] = zeros16
            return 0
        lax.fori_loop(0, C // L, fill_czero, 0)

        rowbase = sid * rows_per_tile
        pltpu.sync_copy(stage_v, ssum.at[pl.ds(rowbase, rows_per_tile)])

        @pl.when(sid == 0)
        def _():
            pltpu.sync_copy(cstage_v, scnt)
        plsc.subcore_barrier()

        base = wid * rows_per_worker

        def chunk(t, _):
            off = base + t * CH
            pltpu.sync_copy(support_hbm.at[pl.ds(off, CH)], rows_v)
            pltpu.sync_copy(labels_hbm.at[pl.ds(off, CH)], labels_v)
            pltpu.sync_copy(rows_v, ssum.at[labels_v], add=True)
            pltpu.sync_copy(ones_v, scnt.at[labels_v], add=True)
            return 0
        lax.fori_loop(0, nchunk, chunk, 0)

        plsc.subcore_barrier()

        pltpu.sync_copy(ssum.at[pl.ds(rowbase, rows_per_tile)], stage_v)
        pltpu.sync_copy(stage_v, sums_out.at[cid, pl.ds(rowbase, rows_per_tile)])

        @pl.when(sid == 0)
        def _():
            pltpu.sync_copy(scnt, cstage_v)
            pltpu.sync_copy(cstage_v, counts_out.at[cid])

    return seg


def _tc_body(ps_ref, pc_ref, q_ref, w_ref, b_ref, out_ref, sp_ref, row_ref):
    i = pl.program_id(0)

    @pl.when(i == 0)
    def _():
        s = ps_ref[0] + ps_ref[1]
        sp = jnp.dot(s, w_ref[:], preferred_element_type=jnp.float32,
                     precision=_HI)
        sp_ref[:] = sp
        ones_row = jnp.ones((1, D), jnp.float32)
        cnt = jnp.sum(pc_ref[:], axis=0, keepdims=True)
        nonempty = cnt > 0.5
        inv = jnp.where(nonempty, 1.0 / jnp.where(nonempty, cnt, 1.0), 0.0)
        msk = jnp.where(nonempty, 1.0, 0.0)
        b_row = b_ref[:]
        sn = lax.dot_general(ones_row, sp * sp, (((1,), (1,)), ((), ())),
                             preferred_element_type=jnp.float32,
                             precision=_HI)
        sb = lax.dot_general(b_row, sp, (((1,), (1,)), ((), ())),
                             preferred_element_type=jnp.float32,
                             precision=_HI)
        bb = jnp.sum(b_row * b_row)
        row_ref[0, :] = inv[0]
        row_ref[1, :] = msk[0]
        row_ref[2, :] = (sn * inv * inv + 2.0 * sb * inv + bb * msk)[0]

    q = q_ref[:]
    e = jnp.dot(q, w_ref[:], preferred_element_type=jnp.float32,
                precision=_HI) + b_ref[:]
    qn = jnp.sum(e * e, axis=1, keepdims=True)
    eb = jnp.sum(e * b_ref[:], axis=1, keepdims=True)
    a = lax.dot_general(e, sp_ref[:], (((1,), (1,)), ((), ())),
                        preferred_element_type=jnp.float32,
                        precision=_HI)
    inv = row_ref[0, :][None, :]
    msk = row_ref[1, :][None, :]
    pn = row_ref[2, :][None, :]
    d2 = qn + pn - 2.0 * (a * inv + eb * msk)
    out_ref[:] = -jnp.sqrt(jnp.maximum(d2, 0.0))


def _tc_distance(psums, pcounts, query, W, b2d):
    nc = psums.shape[0]
    grid = N_QRY // QB
    return pl.pallas_call(
        _tc_body,
        grid=(grid,),
        in_specs=[
            pl.BlockSpec((nc, C, D), lambda i: (0, 0, 0)),
            pl.BlockSpec((nc, C), lambda i: (0, 0)),
            pl.BlockSpec((QB, D), lambda i: (i, 0)),
            pl.BlockSpec((D, D), lambda i: (0, 0)),
            pl.BlockSpec((1, D), lambda i: (0, 0)),
        ],
        out_specs=pl.BlockSpec((QB, C), lambda i: (i, 0)),
        out_shape=jax.ShapeDtypeStruct((N_QRY, C), jnp.float32),
        scratch_shapes=[
            pltpu.VMEM((C, D), jnp.float32),
            pltpu.VMEM((8, C), jnp.float32),
        ],
    )(psums, pcounts, query, W, b2d)


def kernel(support_set, query_set, support_labels, n_way, W, b):
    psums, pcounts = _make_sc_segment_sum()(support_set, support_labels)
    return _tc_distance(psums, pcounts, query_set, W, b.reshape(1, D))


# speedup vs baseline: 12.8281x; 1.2706x over previous
import functools

import jax
import jax.numpy as jnp
from jax import lax
from jax.experimental import pallas as pl
from jax.experimental.pallas import tpu as pltpu
from jax.experimental.pallas import tpu_sc as plsc

N_SUP = 65536
N_QRY = 8192
D = 128
C = 512
L = 16
CH = 128
QB = 1024

_HI = lax.Precision.HIGHEST


def _make_sc_segment_sum():
    mesh = plsc.VectorSubcoreMesh(core_axis_name="c", subcore_axis_name="s")
    nc, ns = mesh.num_cores, mesh.num_subcores
    nw = nc * ns
    rows_per_worker = N_SUP // nw
    nchunk = rows_per_worker // CH
    rows_per_tile = C // ns

    @functools.partial(
        pl.kernel,
        out_type=(
            jax.ShapeDtypeStruct((nc, C, D), jnp.float32),
            jax.ShapeDtypeStruct((nc, C), jnp.float32),
        ),
        mesh=mesh,
        scratch_types=[
            pltpu.VMEM((CH, D), jnp.float32),
            pltpu.VMEM((CH, D), jnp.float32),
            pltpu.VMEM((CH,), jnp.int32),
            pltpu.VMEM((CH,), jnp.int32),
            pltpu.VMEM((CH,), jnp.float32),
            pltpu.VMEM((rows_per_tile, D), jnp.float32),
            pltpu.VMEM((C,), jnp.float32),
            pltpu.VMEM_SHARED((C, D), jnp.float32),
            pltpu.VMEM_SHARED((C,), jnp.float32),
            pltpu.SemaphoreType.DMA,
            pltpu.SemaphoreType.DMA,
            pltpu.SemaphoreType.DMA,
            pltpu.SemaphoreType.DMA,
        ],
    )
    def seg(support_hbm, labels_hbm, sums_out, counts_out,
            rows_a, rows_b, lab_a, lab_b, ones_v, stage_v, cstage_v,
            ssum, scnt, rsem_a, rsem_b, lsem_a, lsem_b):
        cid = lax.axis_index("c")
        sid = lax.axis_index("s")
        wid = sid * nc + cid

        zeros16 = jnp.zeros((L,), jnp.float32)
        ones16 = jnp.ones((L,), jnp.float32)

        def fill_ones(k, _):
            ones_v[pl.ds(k * L, L)] = ones16
            return 0
        lax.fori_loop(0, CH // L, fill_ones, 0)

        def fill_zero(k, _):
            stage_v[k // (D // L), pl.ds((k % (D // L)) * L, L)] = zeros16
            return 0
        lax.fori_loop(0, rows_per_tile * (D // L), fill_zero, 0)

        def fill_czero(k, _):
            cstage_v[pl.ds(k * L, L)] = zeros16
            return 0
        lax.fori_loop(0, C // L, fill_czero, 0)

        rowbase = sid * rows_per_tile
        pltpu.sync_copy(stage_v, ssum.at[pl.ds(rowbase, rows_per_tile)])

        @pl.when(sid == 0)
        def _():
            pltpu.sync_copy(cstage_v, scnt)
        plsc.subcore_barrier()

        base = wid * rows_per_worker
        bufs = ((rows_a, lab_a, rsem_a, lsem_a), (rows_b, lab_b, rsem_b, lsem_b))

        def start_gather(t):
            rv, lv, rs, ls = bufs[t % 2]
            off = base + t * CH
            return (
                pltpu.async_copy(support_hbm.at[pl.ds(off, CH)], rv, rs),
                pltpu.async_copy(labels_hbm.at[pl.ds(off, CH)], lv, ls),
            )

        pending = [start_gather(0), None]
        for t in range(nchunk):
            if t + 1 < nchunk:
                pending[(t + 1) % 2] = start_gather(t + 1)
            for h in pending[t % 2]:
                h.wait()
            rv, lv, _, _ = bufs[t % 2]
            pltpu.sync_copy(rv, ssum.at[lv], add=True)
            pltpu.sync_copy(ones_v, scnt.at[lv], add=True)

        plsc.subcore_barrier()

        pltpu.sync_copy(ssum.at[pl.ds(rowbase, rows_per_tile)], stage_v)
        pltpu.sync_copy(stage_v, sums_out.at[cid, pl.ds(rowbase, rows_per_tile)])

        @pl.when(sid == 0)
        def _():
            pltpu.sync_copy(scnt, cstage_v)
            pltpu.sync_copy(cstage_v, counts_out.at[cid])

    return seg


def _tc_body(ps_ref, pc_ref, q_ref, w_ref, b_ref, out_ref, sp_ref, row_ref):
    i = pl.program_id(0)

    @pl.when(i == 0)
    def _():
        s = ps_ref[0] + ps_ref[1]
        sp = jnp.dot(s, w_ref[:], preferred_element_type=jnp.float32,
                     precision=_HI)
        sp_ref[:] = sp
        ones_row = jnp.ones((1, D), jnp.float32)
        cnt = jnp.sum(pc_ref[:], axis=0, keepdims=True)
        nonempty = cnt > 0.5
        inv = jnp.where(nonempty, 1.0 / jnp.where(nonempty, cnt, 1.0), 0.0)
        msk = jnp.where(nonempty, 1.0, 0.0)
        b_row = b_ref[:]
        sn = lax.dot_general(ones_row, sp * sp, (((1,), (1,)), ((), ())),
                             preferred_element_type=jnp.float32,
                             precision=_HI)
        sb = lax.dot_general(b_row, sp, (((1,), (1,)), ((), ())),
                             preferred_element_type=jnp.float32,
                             precision=_HI)
        bb = jnp.sum(b_row * b_row)
        row_ref[0, :] = inv[0]
        row_ref[1, :] = msk[0]
        row_ref[2, :] = (sn * inv * inv + 2.0 * sb * inv + bb * msk)[0]

    q = q_ref[:]
    e = jnp.dot(q, w_ref[:], preferred_element_type=jnp.float32,
                precision=_HI) + b_ref[:]
    qn = jnp.sum(e * e, axis=1, keepdims=True)
    eb = jnp.sum(e * b_ref[:], axis=1, keepdims=True)
    a = lax.dot_general(e, sp_ref[:], (((1,), (1,)), ((), ())),
                        preferred_element_type=jnp.float32,
                        precision=_HI)
    inv = row_ref[0, :][None, :]
    msk = row_ref[1, :][None, :]
    pn = row_ref[2, :][None, :]
    d2 = qn + pn - 2.0 * (a * inv + eb * msk)
    out_ref[:] = -jnp.sqrt(jnp.maximum(d2, 0.0))


def _tc_distance(psums, pcounts, query, W, b2d):
    nc = psums.shape[0]
    grid = N_QRY // QB
    return pl.pallas_call(
        _tc_body,
        grid=(grid,),
        in_specs=[
            pl.BlockSpec((nc, C, D), lambda i: (0, 0, 0)),
            pl.BlockSpec((nc, C), lambda i: (0, 0)),
            pl.BlockSpec((QB, D), lambda i: (i, 0)),
            pl.BlockSpec((D, D), lambda i: (0, 0)),
            pl.BlockSpec((1, D), lambda i: (0, 0)),
        ],
        out_specs=pl.BlockSpec((QB, C), lambda i: (i, 0)),
        out_shape=jax.ShapeDtypeStruct((N_QRY, C), jnp.float32),
        scratch_shapes=[
            pltpu.VMEM((C, D), jnp.float32),
            pltpu.VMEM((8, C), jnp.float32),
        ],
    )(psums, pcounts, query, W, b2d)


def kernel(support_set, query_set, support_labels, n_way, W, b):
    psums, pcounts = _make_sc_segment_sum()(support_set, support_labels)
    return _tc_distance(psums, pcounts, query_set, W, b.reshape(1, D))


# speedup vs baseline: 16.2086x; 1.2635x over previous
import functools

import jax
import jax.numpy as jnp
from jax import lax
from jax.experimental import pallas as pl
from jax.experimental.pallas import tpu as pltpu
from jax.experimental.pallas import tpu_sc as plsc

N_SUP = 65536
N_QRY = 8192
D = 128
C = 512
L = 16
CH = 128
QB = 1024

_HI = lax.Precision.DEFAULT


def _make_sc_segment_sum():
    mesh = plsc.VectorSubcoreMesh(core_axis_name="c", subcore_axis_name="s")
    nc, ns = mesh.num_cores, mesh.num_subcores
    nw = nc * ns
    rows_per_worker = N_SUP // nw
    nchunk = rows_per_worker // CH
    rows_per_tile = C // ns

    @functools.partial(
        pl.kernel,
        out_type=(
            jax.ShapeDtypeStruct((nc, C, D), jnp.float32),
            jax.ShapeDtypeStruct((nc, C), jnp.float32),
        ),
        mesh=mesh,
        scratch_types=[
            pltpu.VMEM((CH, D), jnp.float32),
            pltpu.VMEM((CH, D), jnp.float32),
            pltpu.VMEM((CH,), jnp.int32),
            pltpu.VMEM((CH,), jnp.int32),
            pltpu.VMEM((CH,), jnp.float32),
            pltpu.VMEM((rows_per_tile, D), jnp.float32),
            pltpu.VMEM((C,), jnp.float32),
            pltpu.VMEM_SHARED((C, D), jnp.float32),
            pltpu.VMEM_SHARED((C,), jnp.float32),
            pltpu.SemaphoreType.DMA,
            pltpu.SemaphoreType.DMA,
            pltpu.SemaphoreType.DMA,
            pltpu.SemaphoreType.DMA,
        ],
    )
    def seg(support_hbm, labels_hbm, sums_out, counts_out,
            rows_a, rows_b, lab_a, lab_b, ones_v, stage_v, cstage_v,
            ssum, scnt, rsem_a, rsem_b, lsem_a, lsem_b):
        cid = lax.axis_index("c")
        sid = lax.axis_index("s")
        wid = sid * nc + cid

        zeros16 = jnp.zeros((L,), jnp.float32)
        ones16 = jnp.ones((L,), jnp.float32)

        def fill_ones(k, _):
            ones_v[pl.ds(k * L, L)] = ones16
            return 0
        lax.fori_loop(0, CH // L, fill_ones, 0)

        def fill_zero(k, _):
            stage_v[k // (D // L), pl.ds((k % (D // L)) * L, L)] = zeros16
            return 0
        lax.fori_loop(0, rows_per_tile * (D // L), fill_zero, 0)

        def fill_czero(k, _):
            cstage_v[pl.ds(k * L, L)] = zeros16
            return 0
        lax.fori_loop(0, C // L, fill_czero, 0)

        rowbase = sid * rows_per_tile
        pltpu.sync_copy(stage_v, ssum.at[pl.ds(rowbase, rows_per_tile)])

        @pl.when(sid == 0)
        def _():
            pltpu.sync_copy(cstage_v, scnt)
        plsc.subcore_barrier()

        base = wid * rows_per_worker
        bufs = ((rows_a, lab_a, rsem_a, lsem_a), (rows_b, lab_b, rsem_b, lsem_b))

        def start_gather(t):
            rv, lv, rs, ls = bufs[t % 2]
            off = base + t * CH
            return (
                pltpu.async_copy(support_hbm.at[pl.ds(off, CH)], rv, rs),
                pltpu.async_copy(labels_hbm.at[pl.ds(off, CH)], lv, ls),
            )

        pending = [start_gather(0), None]
        for t in range(nchunk):
            if t + 1 < nchunk:
                pending[(t + 1) % 2] = start_gather(t + 1)
            for h in pending[t % 2]:
                h.wait()
            rv, lv, _, _ = bufs[t % 2]
            pltpu.sync_copy(rv, ssum.at[lv], add=True)
            pltpu.sync_copy(ones_v, scnt.at[lv], add=True)

        plsc.subcore_barrier()

        pltpu.sync_copy(ssum.at[pl.ds(rowbase, rows_per_tile)], stage_v)
        pltpu.sync_copy(stage_v, sums_out.at[cid, pl.ds(rowbase, rows_per_tile)])

        @pl.when(sid == 0)
        def _():
            pltpu.sync_copy(scnt, cstage_v)
            pltpu.sync_copy(cstage_v, counts_out.at[cid])

    return seg


def _tc_body(ps_ref, pcc_ref, q_ref, w_ref, b_ref, out_ref, proto_ref, pn_ref):
    i = pl.program_id(0)

    @pl.when(i == 0)
    def _():
        s = ps_ref[0] + ps_ref[1]
        sp = jnp.dot(s, w_ref[:], preferred_element_type=jnp.float32,
                     precision=_HI)
        cnt_col = pcc_ref[0] + pcc_ref[1]
        nonempty = cnt_col > 0.5
        inv_col = jnp.where(nonempty,
                            1.0 / jnp.where(nonempty, cnt_col, 1.0), 0.0)
        msk_col = jnp.where(nonempty, 1.0, 0.0)
        proto = sp * inv_col + b_ref[:] * msk_col
        proto_ref[:] = proto
        pn_ref[:] = lax.dot_general(
            jnp.ones((1, D), jnp.float32), proto * proto,
            (((1,), (1,)), ((), ())),
            preferred_element_type=jnp.float32, precision=_HI)

    q = q_ref[:]
    e = jnp.dot(q, w_ref[:], preferred_element_type=jnp.float32,
                precision=_HI) + b_ref[:]
    qn = jnp.sum(e * e, axis=1, keepdims=True)
    a = lax.dot_general(e, proto_ref[:], (((1,), (1,)), ((), ())),
                        preferred_element_type=jnp.float32,
                        precision=_HI)
    d2 = qn + pn_ref[:] - 2.0 * a
    out_ref[:] = -jnp.sqrt(jnp.maximum(d2, 0.0))


def _tc_distance(psums, pcounts, query, W, b2d):
    nc = psums.shape[0]
    grid = N_QRY // QB
    return pl.pallas_call(
        _tc_body,
        grid=(grid,),
        in_specs=[
            pl.BlockSpec((nc, C, D), lambda i: (0, 0, 0)),
            pl.BlockSpec((nc, C, 1), lambda i: (0, 0, 0)),
            pl.BlockSpec((QB, D), lambda i: (i, 0)),
            pl.BlockSpec((D, D), lambda i: (0, 0)),
            pl.BlockSpec((1, D), lambda i: (0, 0)),
        ],
        out_specs=pl.BlockSpec((QB, C), lambda i: (i, 0)),
        out_shape=jax.ShapeDtypeStruct((N_QRY, C), jnp.float32),
        scratch_shapes=[
            pltpu.VMEM((C, D), jnp.float32),
            pltpu.VMEM((1, C), jnp.float32),
        ],
    )(psums, pcounts.reshape(nc, C, 1), query, W, b2d)


def kernel(support_set, query_set, support_labels, n_way, W, b):
    psums, pcounts = _make_sc_segment_sum()(support_set, support_labels)
    return _tc_distance(psums, pcounts, query_set, W, b.reshape(1, D))


# speedup vs baseline: 16.9922x; 1.0483x over previous
import functools

import jax
import jax.numpy as jnp
from jax import lax
from jax.experimental import pallas as pl
from jax.experimental.pallas import tpu as pltpu
from jax.experimental.pallas import tpu_sc as plsc

N_SUP = 65536
N_QRY = 8192
D = 128
C = 512
L = 16
CH = 128
QB = 1024

_HI = lax.Precision.DEFAULT


def _make_sc_segment_sum():
    mesh = plsc.VectorSubcoreMesh(core_axis_name="c", subcore_axis_name="s")
    nc, ns = mesh.num_cores, mesh.num_subcores
    nw = nc * ns
    rows_per_worker = N_SUP // nw
    nchunk = rows_per_worker // CH
    rows_per_tile = C // ns

    @functools.partial(
        pl.kernel,
        out_type=(
            jax.ShapeDtypeStruct((nc, C, D), jnp.float32),
            jax.ShapeDtypeStruct((nc, C), jnp.float32),
        ),
        mesh=mesh,
        scratch_types=[
            pltpu.VMEM((CH, D), jnp.float32),
            pltpu.VMEM((CH, D), jnp.float32),
            pltpu.VMEM((CH,), jnp.int32),
            pltpu.VMEM((CH,), jnp.int32),
            pltpu.VMEM((CH,), jnp.float32),
            pltpu.VMEM((rows_per_tile, D), jnp.float32),
            pltpu.VMEM((C,), jnp.float32),
            pltpu.VMEM_SHARED((C, D), jnp.float32),
            pltpu.VMEM_SHARED((C,), jnp.float32),
            pltpu.SemaphoreType.DMA,
            pltpu.SemaphoreType.DMA,
            pltpu.SemaphoreType.DMA,
            pltpu.SemaphoreType.DMA,
        ],
    )
    def seg(support_hbm, labels_hbm, sums_out, counts_out,
            rows_a, rows_b, lab_a, lab_b, ones_v, stage_v, cstage_v,
            ssum, scnt, rsem_a, rsem_b, lsem_a, lsem_b):
        cid = lax.axis_index("c")
        sid = lax.axis_index("s")
        wid = sid * nc + cid

        zeros16 = jnp.zeros((L,), jnp.float32)
        ones16 = jnp.ones((L,), jnp.float32)

        def fill_ones(k, _):
            ones_v[pl.ds(k * L, L)] = ones16
            return 0
        lax.fori_loop(0, CH // L, fill_ones, 0)

        def fill_zero(k, _):
            stage_v[k // (D // L), pl.ds((k % (D // L)) * L, L)] = zeros16
            return 0
        lax.fori_loop(0, rows_per_tile * (D // L), fill_zero, 0)

        def fill_czero(k, _):
            cstage_v[pl.ds(k * L, L)] = zeros16
            return 0
        lax.fori_loop(0, C // L, fill_czero, 0)

        rowbase = sid * rows_per_tile
        pltpu.sync_copy(stage_v, ssum.at[pl.ds(rowbase, rows_per_tile)])

        @pl.when(sid == 0)
        def _():
            pltpu.sync_copy(cstage_v, scnt)
        plsc.subcore_barrier()

        base = wid * rows_per_worker
        bufs = ((rows_a, lab_a, rsem_a, lsem_a), (rows_b, lab_b, rsem_b, lsem_b))

        def start_gather(t):
            rv, lv, rs, ls = bufs[t % 2]
            off = base + t * CH
            pltpu.async_copy(support_hbm.at[pl.ds(off, CH)], rv, rs)
            pltpu.async_copy(labels_hbm.at[pl.ds(off, CH)], lv, ls)

        start_gather(0)
        start_gather(1)

        def ring(k, _):
            for bslot in range(2):
                t = 2 * k + bslot
                rv, lv, rs, ls = bufs[bslot]
                off = base + t * CH
                pltpu.make_async_copy(
                    support_hbm.at[pl.ds(off, CH)], rv, rs).wait()
                pltpu.make_async_copy(
                    labels_hbm.at[pl.ds(off, CH)], lv, ls).wait()
                pltpu.sync_copy(rv, ssum.at[lv], add=True)
                pltpu.sync_copy(ones_v, scnt.at[lv], add=True)

                @pl.when(t + 2 < nchunk)
                def _():
                    off2 = base + (t + 2) * CH
                    pltpu.async_copy(support_hbm.at[pl.ds(off2, CH)], rv, rs)
                    pltpu.async_copy(labels_hbm.at[pl.ds(off2, CH)], lv, ls)
            return 0
        lax.fori_loop(0, nchunk // 2, ring, 0)

        plsc.subcore_barrier()

        pltpu.sync_copy(ssum.at[pl.ds(rowbase, rows_per_tile)], stage_v)
        pltpu.sync_copy(stage_v, sums_out.at[cid, pl.ds(rowbase, rows_per_tile)])

        @pl.when(sid == 0)
        def _():
            pltpu.sync_copy(scnt, cstage_v)
            pltpu.sync_copy(cstage_v, counts_out.at[cid])

    return seg


def _tc_body(ps_ref, pcc_ref, q_ref, w_ref, b_ref, out_ref, proto_ref, pn_ref):
    i = pl.program_id(0)

    @pl.when(i == 0)
    def _():
        s = ps_ref[0] + ps_ref[1]
        sp = jnp.dot(s, w_ref[:], preferred_element_type=jnp.float32,
                     precision=_HI)
        cnt_row = jnp.sum(pcc_ref[:], axis=0, keepdims=True)
        cnt_col = jnp.transpose(cnt_row, (1, 0))
        nonempty = cnt_col > 0.5
        inv_col = jnp.where(nonempty,
                            1.0 / jnp.where(nonempty, cnt_col, 1.0), 0.0)
        msk_col = jnp.where(nonempty, 1.0, 0.0)
        proto = sp * inv_col + b_ref[:] * msk_col
        pn_ref[:] = lax.dot_general(
            jnp.ones((1, D), jnp.float32), proto * proto,
            (((1,), (1,)), ((), ())),
            preferred_element_type=jnp.float32, precision=_HI)
        proto_ref[:] = -2.0 * proto

    q = q_ref[:]
    e = jnp.dot(q, w_ref[:], preferred_element_type=jnp.float32,
                precision=_HI) + b_ref[:]
    qn = jnp.sum(e * e, axis=1, keepdims=True)
    a2 = lax.dot_general(e, proto_ref[:], (((1,), (1,)), ((), ())),
                         preferred_element_type=jnp.float32,
                         precision=_HI)
    d2 = (qn + pn_ref[:]) + a2
    out_ref[:] = -jnp.sqrt(jnp.maximum(d2, 0.0))


def _tc_distance(psums, pcounts, query, W, b2d):
    nc = psums.shape[0]
    grid = N_QRY // QB
    return pl.pallas_call(
        _tc_body,
        grid=(grid,),
        in_specs=[
            pl.BlockSpec((nc, C, D), lambda i: (0, 0, 0)),
            pl.BlockSpec((nc, C), lambda i: (0, 0)),
            pl.BlockSpec((QB, D), lambda i: (i, 0)),
            pl.BlockSpec((D, D), lambda i: (0, 0)),
            pl.BlockSpec((1, D), lambda i: (0, 0)),
        ],
        out_specs=pl.BlockSpec((QB, C), lambda i: (i, 0)),
        out_shape=jax.ShapeDtypeStruct((N_QRY, C), jnp.float32),
        scratch_shapes=[
            pltpu.VMEM((C, D), jnp.float32),
            pltpu.VMEM((1, C), jnp.float32),
        ],
    )(psums, pcounts, query, W, b2d)


def kernel(support_set, query_set, support_labels, n_way, W, b):
    psums, pcounts = _make_sc_segment_sum()(support_set, support_labels)
    return _tc_distance(psums, pcounts, query_set, W, b.reshape(1, D))


# speedup vs baseline: 17.8451x; 1.0502x over previous
import functools

import jax
import jax.numpy as jnp
from jax import lax
from jax.experimental import pallas as pl
from jax.experimental.pallas import tpu as pltpu
from jax.experimental.pallas import tpu_sc as plsc

N_SUP = 65536
N_QRY = 8192
D = 128
C = 512
L = 16
CH = 128
QB = 2048

_HI = lax.Precision.DEFAULT


def _make_sc_segment_sum():
    mesh = plsc.VectorSubcoreMesh(core_axis_name="c", subcore_axis_name="s")
    nc, ns = mesh.num_cores, mesh.num_subcores
    nw = nc * ns
    rows_per_worker = N_SUP // nw
    nchunk = rows_per_worker // CH
    rows_per_tile = C // ns

    @functools.partial(
        pl.kernel,
        out_type=(
            jax.ShapeDtypeStruct((nc, C, D), jnp.float32),
            jax.ShapeDtypeStruct((nc, C), jnp.float32),
        ),
        mesh=mesh,
        scratch_types=[
            pltpu.VMEM((CH, D), jnp.float32),
            pltpu.VMEM((CH, D), jnp.float32),
            pltpu.VMEM((CH,), jnp.int32),
            pltpu.VMEM((CH,), jnp.int32),
            pltpu.VMEM((CH,), jnp.float32),
            pltpu.VMEM((rows_per_tile, D), jnp.float32),
            pltpu.VMEM((C,), jnp.float32),
            pltpu.VMEM_SHARED((C, D), jnp.float32),
            pltpu.VMEM_SHARED((C,), jnp.float32),
            pltpu.SemaphoreType.DMA,
            pltpu.SemaphoreType.DMA,
            pltpu.SemaphoreType.DMA,
            pltpu.SemaphoreType.DMA,
        ],
    )
    def seg(support_hbm, labels_hbm, sums_out, counts_out,
            rows_a, rows_b, lab_a, lab_b, ones_v, stage_v, cstage_v,
            ssum, scnt, rsem_a, rsem_b, lsem_a, lsem_b):
        cid = lax.axis_index("c")
        sid = lax.axis_index("s")
        wid = sid * nc + cid

        zeros16 = jnp.zeros((L,), jnp.float32)
        ones16 = jnp.ones((L,), jnp.float32)

        base = wid * rows_per_worker
        bufs = ((rows_a, lab_a, rsem_a, lsem_a), (rows_b, lab_b, rsem_b, lsem_b))

        def start_gather(t):
            rv, lv, rs, ls = bufs[t % 2]
            off = base + t * CH
            pltpu.async_copy(support_hbm.at[pl.ds(off, CH)], rv, rs)
            pltpu.async_copy(labels_hbm.at[pl.ds(off, CH)], lv, ls)

        start_gather(0)
        start_gather(1)

        def fill_ones(k, _):
            ones_v[pl.ds(k * L, L)] = ones16
            return 0
        lax.fori_loop(0, CH // L, fill_ones, 0)

        def fill_zero(k, _):
            stage_v[k // (D // L), pl.ds((k % (D // L)) * L, L)] = zeros16
            return 0
        lax.fori_loop(0, rows_per_tile * (D // L), fill_zero, 0)

        def fill_czero(k, _):
            cstage_v[pl.ds(k * L, L)] = zeros16
            return 0
        lax.fori_loop(0, C // L, fill_czero, 0)

        rowbase = sid * rows_per_tile
        pltpu.sync_copy(stage_v, ssum.at[pl.ds(rowbase, rows_per_tile)])

        @pl.when(sid == 0)
        def _():
            pltpu.sync_copy(cstage_v, scnt)
        plsc.subcore_barrier()

        def ring(k, _):
            for bslot in range(2):
                t = 2 * k + bslot
                rv, lv, rs, ls = bufs[bslot]
                off = base + t * CH
                pltpu.make_async_copy(
                    support_hbm.at[pl.ds(off, CH)], rv, rs).wait()
                pltpu.make_async_copy(
                    labels_hbm.at[pl.ds(off, CH)], lv, ls).wait()
                pltpu.sync_copy(rv, ssum.at[lv], add=True)
                pltpu.sync_copy(ones_v, scnt.at[lv], add=True)

                @pl.when(t + 2 < nchunk)
                def _():
                    off2 = base + (t + 2) * CH
                    pltpu.async_copy(support_hbm.at[pl.ds(off2, CH)], rv, rs)
                    pltpu.async_copy(labels_hbm.at[pl.ds(off2, CH)], lv, ls)
            return 0
        lax.fori_loop(0, nchunk // 2, ring, 0)

        plsc.subcore_barrier()

        pltpu.sync_copy(ssum.at[pl.ds(rowbase, rows_per_tile)], stage_v)
        pltpu.sync_copy(stage_v, sums_out.at[cid, pl.ds(rowbase, rows_per_tile)])

        @pl.when(sid == 0)
        def _():
            pltpu.sync_copy(scnt, cstage_v)
            pltpu.sync_copy(cstage_v, counts_out.at[cid])

    return seg


def _tc_body(ps_ref, pcc_ref, q_ref, w_ref, b_ref, out_ref, proto_ref, pn_ref):
    i = pl.program_id(0)

    @pl.when(i == 0)
    def _():
        s = ps_ref[0] + ps_ref[1]
        sp = jnp.dot(s, w_ref[:], preferred_element_type=jnp.float32,
                     precision=_HI)
        cnt_row = jnp.sum(pcc_ref[:], axis=0, keepdims=True)
        cnt_col = jnp.transpose(cnt_row, (1, 0))
        nonempty = cnt_col > 0.5
        inv_col = jnp.where(nonempty,
                            1.0 / jnp.where(nonempty, cnt_col, 1.0), 0.0)
        msk_col = jnp.where(nonempty, 1.0, 0.0)
        proto = sp * inv_col + b_ref[:] * msk_col
        pn_ref[:] = lax.dot_general(
            jnp.ones((1, D), jnp.float32), proto * proto,
            (((1,), (1,)), ((), ())),
            preferred_element_type=jnp.float32, precision=_HI)
        proto_ref[:] = -2.0 * proto

    q = q_ref[:]
    e = jnp.dot(q, w_ref[:], preferred_element_type=jnp.float32,
                precision=_HI) + b_ref[:]
    qn = jnp.sum(e * e, axis=1, keepdims=True)
    a2 = lax.dot_general(e, proto_ref[:], (((1,), (1,)), ((), ())),
                         preferred_element_type=jnp.float32,
                         precision=_HI)
    d2 = (qn + pn_ref[:]) + a2
    out_ref[:] = -jnp.sqrt(jnp.maximum(d2, 0.0))


def _tc_distance(psums, pcounts, query, W, b2d):
    nc = psums.shape[0]
    grid = N_QRY // QB
    return pl.pallas_call(
        _tc_body,
        grid=(grid,),
        in_specs=[
            pl.BlockSpec((nc, C, D), lambda i: (0, 0, 0)),
            pl.BlockSpec((nc, C), lambda i: (0, 0)),
            pl.BlockSpec((QB, D), lambda i: (i, 0)),
            pl.BlockSpec((D, D), lambda i: (0, 0)),
            pl.BlockSpec((1, D), lambda i: (0, 0)),
        ],
        out_specs=pl.BlockSpec((QB, C), lambda i: (i, 0)),
        out_shape=jax.ShapeDtypeStruct((N_QRY, C), jnp.float32),
        scratch_shapes=[
            pltpu.VMEM((C, D), jnp.float32),
            pltpu.VMEM((1, C), jnp.float32),
        ],
    )(psums, pcounts, query, W, b2d)


def kernel(support_set, query_set, support_labels, n_way, W, b):
    psums, pcounts = _make_sc_segment_sum()(support_set, support_labels)
    return _tc_distance(psums, pcounts, query_set, W, b.reshape(1, D))


# speedup vs baseline: 17.9445x; 1.0056x over previous
import functools

import jax
import jax.numpy as jnp
from jax import lax
from jax.experimental import pallas as pl
from jax.experimental.pallas import tpu as pltpu
from jax.experimental.pallas import tpu_sc as plsc

N_SUP = 65536
N_QRY = 8192
D = 128
C = 512
L = 16
CH = 128
QB = 2048

_HI = lax.Precision.DEFAULT


def _make_sc_segment_sum():
    mesh = plsc.VectorSubcoreMesh(core_axis_name="c", subcore_axis_name="s")
    nc, ns = mesh.num_cores, mesh.num_subcores
    nw = nc * ns
    rows_per_worker = N_SUP // nw
    nchunk = rows_per_worker // CH
    rows_per_tile = C // ns

    @functools.partial(
        pl.kernel,
        out_type=(
            jax.ShapeDtypeStruct((nc, C, D), jnp.float32),
            jax.ShapeDtypeStruct((nc, C), jnp.float32),
        ),
        mesh=mesh,
        scratch_types=[
            pltpu.VMEM((CH, D), jnp.float32),
            pltpu.VMEM((CH, D), jnp.float32),
            pltpu.VMEM((CH,), jnp.int32),
            pltpu.VMEM((CH,), jnp.int32),
            pltpu.VMEM((CH,), jnp.float32),
            pltpu.VMEM((rows_per_tile, D), jnp.float32),
            pltpu.VMEM((C,), jnp.float32),
            pltpu.VMEM_SHARED((C, D), jnp.float32),
            pltpu.VMEM_SHARED((C,), jnp.float32),
            pltpu.SemaphoreType.DMA,
            pltpu.SemaphoreType.DMA,
            pltpu.SemaphoreType.DMA,
            pltpu.SemaphoreType.DMA,
            pltpu.SemaphoreType.DMA,
        ],
    )
    def seg(support_hbm, labels_hbm, sums_out, counts_out,
            rows_a, rows_b, lab_a, lab_b, ones_v, stage_v, cstage_v,
            ssum, scnt, rsem_a, rsem_b, lsem_a, lsem_b, csem):
        cid = lax.axis_index("c")
        sid = lax.axis_index("s")
        wid = sid * nc + cid

        zeros16 = jnp.zeros((L,), jnp.float32)
        ones16 = jnp.ones((L,), jnp.float32)

        base = wid * rows_per_worker
        bufs = ((rows_a, lab_a, rsem_a, lsem_a), (rows_b, lab_b, rsem_b, lsem_b))

        def start_gather(t):
            rv, lv, rs, ls = bufs[t % 2]
            off = base + t * CH
            pltpu.async_copy(support_hbm.at[pl.ds(off, CH)], rv, rs)
            pltpu.async_copy(labels_hbm.at[pl.ds(off, CH)], lv, ls)

        start_gather(0)
        start_gather(1)

        def fill_ones(k, _):
            ones_v[pl.ds(k * L, L)] = ones16
            return 0
        lax.fori_loop(0, CH // L, fill_ones, 0)

        def fill_zero(k, _):
            for j in range(D // L):
                stage_v[k, pl.ds(j * L, L)] = zeros16
            return 0
        lax.fori_loop(0, rows_per_tile, fill_zero, 0)

        def fill_czero(k, _):
            cstage_v[pl.ds(k * L, L)] = zeros16
            return 0
        lax.fori_loop(0, C // L, fill_czero, 0)

        rowbase = sid * rows_per_tile
        pltpu.sync_copy(stage_v, ssum.at[pl.ds(rowbase, rows_per_tile)])

        @pl.when(sid == 0)
        def _():
            pltpu.sync_copy(cstage_v, scnt)
        plsc.subcore_barrier()

        def ring(k, _):
            for bslot in range(2):
                t = 2 * k + bslot
                rv, lv, rs, ls = bufs[bslot]
                off = base + t * CH
                pltpu.make_async_copy(
                    support_hbm.at[pl.ds(off, CH)], rv, rs).wait()
                pltpu.make_async_copy(
                    labels_hbm.at[pl.ds(off, CH)], lv, ls).wait()
                cnt_h = pltpu.async_copy(ones_v, scnt.at[lv], csem, add=True)
                pltpu.sync_copy(rv, ssum.at[lv], add=True)
                cnt_h.wait()

                @pl.when(t + 2 < nchunk)
                def _():
                    off2 = base + (t + 2) * CH
                    pltpu.async_copy(support_hbm.at[pl.ds(off2, CH)], rv, rs)
                    pltpu.async_copy(labels_hbm.at[pl.ds(off2, CH)], lv, ls)
            return 0
        lax.fori_loop(0, nchunk // 2, ring, 0)

        plsc.subcore_barrier()

        pltpu.sync_copy(ssum.at[pl.ds(rowbase, rows_per_tile)], stage_v)
        pltpu.sync_copy(stage_v, sums_out.at[cid, pl.ds(rowbase, rows_per_tile)])

        @pl.when(sid == 0)
        def _():
            pltpu.sync_copy(scnt, cstage_v)
            pltpu.sync_copy(cstage_v, counts_out.at[cid])

    return seg


def _tc_body(ps_ref, pcc_ref, q_ref, w_ref, b_ref, out_ref, proto_ref, pn_ref):
    i = pl.program_id(0)

    @pl.when(i == 0)
    def _():
        s = ps_ref[0] + ps_ref[1]
        sp = jnp.dot(s, w_ref[:], preferred_element_type=jnp.float32,
                     precision=_HI)
        cnt_row = jnp.sum(pcc_ref[:], axis=0, keepdims=True)
        cnt_col = jnp.transpose(cnt_row, (1, 0))
        nonempty = cnt_col > 0.5
        inv_col = jnp.where(nonempty,
                            1.0 / jnp.where(nonempty, cnt_col, 1.0), 0.0)
        msk_col = jnp.where(nonempty, 1.0, 0.0)
        proto = sp * inv_col + b_ref[:] * msk_col
        pn_ref[:] = lax.dot_general(
            jnp.ones((1, D), jnp.float32), proto * proto,
            (((1,), (1,)), ((), ())),
            preferred_element_type=jnp.float32, precision=_HI)
        proto_ref[:] = -2.0 * proto

    q = q_ref[:]
    e = jnp.dot(q, w_ref[:], preferred_element_type=jnp.float32,
                precision=_HI) + b_ref[:]
    qn = jnp.sum(e * e, axis=1, keepdims=True)
    a2 = lax.dot_general(e, proto_ref[:], (((1,), (1,)), ((), ())),
                         preferred_element_type=jnp.float32,
                         precision=_HI)
    d2 = (qn + pn_ref[:]) + a2
    out_ref[:] = -jnp.sqrt(jnp.maximum(d2, 0.0))


def _tc_distance(psums, pcounts, query, W, b2d):
    nc = psums.shape[0]
    grid = N_QRY // QB
    return pl.pallas_call(
        _tc_body,
        grid=(grid,),
        in_specs=[
            pl.BlockSpec((nc, C, D), lambda i: (0, 0, 0)),
            pl.BlockSpec((nc, C), lambda i: (0, 0)),
            pl.BlockSpec((QB, D), lambda i: (i, 0)),
            pl.BlockSpec((D, D), lambda i: (0, 0)),
            pl.BlockSpec((1, D), lambda i: (0, 0)),
        ],
        out_specs=pl.BlockSpec((QB, C), lambda i: (i, 0)),
        out_shape=jax.ShapeDtypeStruct((N_QRY, C), jnp.float32),
        scratch_shapes=[
            pltpu.VMEM((C, D), jnp.float32),
            pltpu.VMEM((1, C), jnp.float32),
        ],
    )(psums, pcounts, query, W, b2d)


def kernel(support_set, query_set, support_labels, n_way, W, b):
    psums, pcounts = _make_sc_segment_sum()(support_set, support_labels)
    return _tc_distance(psums, pcounts, query_set, W, b.reshape(1, D))
